# Initial kernel scaffold; baseline (speedup 1.0000x reference)
import jax
import jax.numpy as jnp
from jax.experimental import pallas as pl


def kernel(x, W_patch, b_patch, cls_token, pos_embed, ln_local_g, ln_local_b, W_local, b_local, ln_cls_g, ln_cls_b, W_cls, b_cls, Wo1, bo1, Wo2, bo2, Wo3, bo3, ln_m_g, ln_m_b, W_a, W_g, W_out):
    raise NotImplementedError("write your pallas kernel here")


# speedup vs baseline: 1.6980x; 1.6980x over previous
import functools

import jax
import jax.numpy as jnp
from jax.experimental import pallas as pl

B, Cin, H, P, D = 16, 3, 384, 16, 768
G = H // P
N = G * G
BLK = N
TOK_STEPS = B * N // BLK
GRID = TOK_STEPS + 1


def _gelu_exact(x):
    return x * 0.5 * (1.0 + jax.lax.erf(x * 0.7071067811865476))


def _ln_rows(x, g, b, eps=1e-5):
    m = jnp.mean(x, axis=-1, keepdims=True)
    v = jnp.mean((x - m) ** 2, axis=-1, keepdims=True)
    return (x - m) / jnp.sqrt(v + eps) * g + b


def _fused_body(xp_ref, pos_ref, cls_t_ref,
                WpT_ref, b_patch_ref,
                lnl_g_ref, lnl_b_ref, W_local_ref, b_local_ref,
                lnc_g_ref, lnc_b_ref, W_cls_ref, b_cls_ref,
                Wo1t_ref, Wo1b_ref, bo1_ref,
                Wo2_ref, bo2_ref, wd_ref, bd_ref,
                lnm_g_ref, lnm_b_ref, W_ag_ref, W_out_ref,
                out_ref):
    s = pl.program_id(0)
    dot = functools.partial(jnp.dot, preferred_element_type=jnp.float32)

    t = dot(xp_ref[...], WpT_ref[...]) + b_patch_ref[...] + pos_ref[...]

    hn_l = _ln_rows(t, lnl_g_ref[...], lnl_b_ref[...])
    lx = _gelu_exact(dot(hn_l, W_local_ref[...]) + b_local_ref[...])

    cls_t = cls_t_ref[...]
    hn_c = _ln_rows(cls_t, lnc_g_ref[...], lnc_b_ref[...])
    cls_f = _gelu_exact(dot(hn_c, W_cls_ref[...]) + b_cls_ref[...])
    zc = dot(cls_f, Wo1b_ref[...]) + bo1_ref[...]

    z1 = _gelu_exact(dot(lx, Wo1t_ref[...]) + zc)
    z2 = _gelu_exact(dot(z1, Wo2_ref[...]) + bo2_ref[...])
    logit = dot(z2, wd_ref[...]) + bd_ref[...]
    p = jax.nn.sigmoid(logit)
    p = jnp.where(s == TOK_STEPS, jnp.ones_like(p), p)

    m = t * p

    hn = _ln_rows(m, lnm_g_ref[...], lnm_b_ref[...])
    ag = dot(hn, W_ag_ref[...])
    a = ag[:, : 2 * D]
    g = ag[:, 2 * D:]
    mix = dot(a * jax.nn.silu(g), W_out_ref[...])
    out_ref[...] = m + mix


def kernel(x, W_patch, b_patch, cls_token, pos_embed,
           ln_local_g, ln_local_b, W_local, b_local,
           ln_cls_g, ln_cls_b, W_cls, b_cls,
           Wo1, bo1, Wo2, bo2, Wo3, bo3,
           ln_m_g, ln_m_b, W_a, W_g, W_out):
    f32 = jnp.float32

    xp = (x.reshape(B, Cin, G, P, G, P)
            .transpose(0, 2, 4, 1, 3, 5)
            .reshape(B * N, Cin * P * P))
    xp = jnp.pad(xp, ((0, BLK), (0, 0)))

    cls_t = (cls_token[0] + pos_embed[0, 0:1]).astype(f32)
    pos_tok = pos_embed[0, 1:, :]
    cls_pos_row = cls_t - b_patch[None, :]
    pos_cls_blk = jnp.concatenate(
        [cls_pos_row, jnp.zeros((BLK - 1, D), f32)], axis=0)
    pos_all = jnp.concatenate([pos_tok, pos_cls_blk], axis=0)

    WpT = W_patch.T
    Wo1t = Wo1[: D // 2]
    Wo1b = Wo1[D // 2:]
    wd = Wo3[:, 0:1] - Wo3[:, 1:2]
    bd_arr = (bo3[0] - bo3[1]).reshape(1, 1)
    W_ag = jnp.concatenate([W_a, W_g], axis=1)

    row = lambda v: v.reshape(1, -1)
    full = lambda shape: pl.BlockSpec(shape, lambda s: (0, 0))

    out_flat = pl.pallas_call(
        _fused_body,
        grid=(GRID,),
        in_specs=[
            pl.BlockSpec((BLK, Cin * P * P), lambda s: (s, 0)),
            pl.BlockSpec((BLK, D), lambda s: (s // TOK_STEPS, 0)),
            full((1, D)),
            full((Cin * P * P, D)),
            full((1, D)),
            full((1, D)), full((1, D)),
            full((D, D // 2)), full((1, D // 2)),
            full((1, D)), full((1, D)),
            full((D, D // 2)), full((1, D // 2)),
            full((D // 2, D // 2)), full((D // 2, D // 2)),
            full((1, D // 2)),
            full((D // 2, D // 4)), full((1, D // 4)),
            full((D // 4, 1)),
            full((1, 1)),
            full((1, D)), full((1, D)),
            full((D, 4 * D)),
            full((2 * D, D)),
        ],
        out_specs=pl.BlockSpec((BLK, D), lambda s: (s, 0)),
        out_shape=jax.ShapeDtypeStruct((GRID * BLK, D), f32),
    )(
        xp, pos_all, cls_t,
        WpT, row(b_patch),
        row(ln_local_g), row(ln_local_b), W_local, row(b_local),
        row(ln_cls_g), row(ln_cls_b), W_cls, row(b_cls),
        Wo1t, Wo1b, row(bo1),
        Wo2, row(bo2), wd, bd_arr,
        row(ln_m_g), row(ln_m_b), W_ag, W_out,
    )

    tok_out = out_flat[: B * N].reshape(B, N, D)
    cls_o = jnp.broadcast_to(out_flat[B * N: B * N + 1][None], (B, 1, D))
    return jnp.concatenate([cls_o, tok_out], axis=1)


# speedup vs baseline: 1.8500x; 1.0896x over previous
import functools

import jax
import jax.numpy as jnp
from jax.experimental import pallas as pl

B, Cin, H, P, D = 16, 3, 384, 16, 768
G = H // P
N = G * G
R = N + 1


def _gelu_exact(x):
    return x * 0.5 * (1.0 + jax.lax.erf(x * 0.7071067811865476))


def _ln_rows(x, g, b, eps=1e-5):
    m = jnp.mean(x, axis=-1, keepdims=True)
    v = jnp.mean((x - m) ** 2, axis=-1, keepdims=True)
    return (x - m) / jnp.sqrt(v + eps) * g + b


def _fused_body(xp_ref, pos_ref,
                WpT_ref, b_patch_ref,
                lnl_g_ref, lnl_b_ref, W_local_ref, b_local_ref,
                lnc_g_ref, lnc_b_ref, W_cls_ref, b_cls_ref,
                Wo1t_ref, Wo1b_ref, bo1_ref,
                Wo2_ref, bo2_ref, wd_ref, bd_ref,
                lnm_g_ref, lnm_b_ref, W_ag_ref, W_out_ref,
                out_ref):
    dot = functools.partial(jnp.dot, preferred_element_type=jnp.float32)

    xb = xp_ref[0]
    t = dot(xb, WpT_ref[...]) + b_patch_ref[...] + pos_ref[...]

    hn_l = _ln_rows(t, lnl_g_ref[...], lnl_b_ref[...])
    lx = _gelu_exact(dot(hn_l, W_local_ref[...]) + b_local_ref[...])

    cls_t = t[0:1]
    hn_c = _ln_rows(cls_t, lnc_g_ref[...], lnc_b_ref[...])
    cls_f = _gelu_exact(dot(hn_c, W_cls_ref[...]) + b_cls_ref[...])
    zc = dot(cls_f, Wo1b_ref[...]) + bo1_ref[...]

    z1 = _gelu_exact(dot(lx, Wo1t_ref[...]) + zc)
    z2 = _gelu_exact(dot(z1, Wo2_ref[...]) + bo2_ref[...])
    logit = dot(z2, wd_ref[...]) + bd_ref[...]
    p = jax.nn.sigmoid(logit)
    rows = jax.lax.broadcasted_iota(jnp.int32, (R, 1), 0)
    p = jnp.where(rows == 0, jnp.ones_like(p), p)

    m = t * p

    hn = _ln_rows(m, lnm_g_ref[...], lnm_b_ref[...])
    ag = dot(hn, W_ag_ref[...])
    a = ag[:, : 2 * D]
    g = ag[:, 2 * D:]
    mix = dot(a * jax.nn.silu(g), W_out_ref[...])
    out_ref[0] = m + mix


def kernel(x, W_patch, b_patch, cls_token, pos_embed,
           ln_local_g, ln_local_b, W_local, b_local,
           ln_cls_g, ln_cls_b, W_cls, b_cls,
           Wo1, bo1, Wo2, bo2, Wo3, bo3,
           ln_m_g, ln_m_b, W_a, W_g, W_out):
    f32 = jnp.float32

    xp = (x.reshape(B, Cin, G, P, G, P)
            .transpose(0, 2, 4, 1, 3, 5)
            .reshape(B, N, Cin * P * P))
    xp = jnp.pad(xp, ((0, 0), (1, 0), (0, 0)))

    cls_t = (cls_token[0] + pos_embed[0, 0:1]).astype(f32)
    cls_pos_row = cls_t - b_patch[None, :]
    pos_all = jnp.concatenate([cls_pos_row, pos_embed[0, 1:, :]], axis=0)

    WpT = W_patch.T
    Wo1t = Wo1[: D // 2]
    Wo1b = Wo1[D // 2:]
    wd = Wo3[:, 0:1] - Wo3[:, 1:2]
    bd_arr = (bo3[0] - bo3[1]).reshape(1, 1)
    W_ag = jnp.concatenate([W_a, W_g], axis=1)

    row = lambda v: v.reshape(1, -1)
    full = lambda shape: pl.BlockSpec(shape, lambda b: (0,) * len(shape))

    out = pl.pallas_call(
        _fused_body,
        grid=(B,),
        in_specs=[
            pl.BlockSpec((1, R, Cin * P * P), lambda b: (b, 0, 0)),
            full((R, D)),
            full((Cin * P * P, D)),
            full((1, D)),
            full((1, D)), full((1, D)),
            full((D, D // 2)), full((1, D // 2)),
            full((1, D)), full((1, D)),
            full((D, D // 2)), full((1, D // 2)),
            full((D // 2, D // 2)), full((D // 2, D // 2)),
            full((1, D // 2)),
            full((D // 2, D // 4)), full((1, D // 4)),
            full((D // 4, 1)),
            full((1, 1)),
            full((1, D)), full((1, D)),
            full((D, 4 * D)),
            full((2 * D, D)),
        ],
        out_specs=pl.BlockSpec((1, R, D), lambda b: (b, 0, 0)),
        out_shape=jax.ShapeDtypeStruct((B, R, D), f32),
    )(
        xp, pos_all,
        WpT, row(b_patch),
        row(ln_local_g), row(ln_local_b), W_local, row(b_local),
        row(ln_cls_g), row(ln_cls_b), W_cls, row(b_cls),
        Wo1t, Wo1b, row(bo1),
        Wo2, row(bo2), wd, bd_arr,
        row(ln_m_g), row(ln_m_b), W_ag, W_out,
    )
    return out
